# baseline (speedup 1.0000x reference)
import jax, jax.numpy as jnp
from jax.experimental import pallas as pl

def _gcn_layer(x, src, dst, norm, W, b):
    xw = x @ W
    msg = jnp.take(xw, src, axis=0) * norm[:, None]
    out = jnp.zeros((x.shape[0], W.shape[1]), dtype=x.dtype).at[dst].add(msg)
    return out + b

def kernel(x, edge_index, batch, W1, b1, W2, b2, W3, b3):
    n = x.shape[0]
    loop = jnp.arange(n, dtype=edge_index.dtype)
    src = jnp.concatenate([edge_index[0], loop])
    dst = jnp.concatenate([edge_index[1], loop])
    deg = jnp.zeros((n,), dtype=x.dtype).at[dst].add(1.0)
    dinv = jnp.where(deg > 0, 1.0 / jnp.sqrt(deg), 0.0)
    norm = jnp.take(dinv, src) * jnp.take(dinv, dst)
    h = jax.nn.relu(_gcn_layer(x, src, dst, norm, W1, b1))
    h = jax.nn.relu(_gcn_layer(h, src, dst, norm, W2, b2))
    h = _gcn_layer(h, src, dst, norm, W3, b3)
    sums = jax.ops.segment_sum(h, batch, num_segments=128)
    cnt = jax.ops.segment_sum(jnp.ones((n,), dtype=h.dtype), batch, num_segments=128)
    return sums / jnp.maximum(cnt, 1.0)[:, None]


# speedup vs baseline: 1.3244x; 1.3244x over previous
import functools

import jax
import jax.numpy as jnp
from jax import lax
from jax.experimental import pallas as pl
from jax.experimental.pallas import tpu as pltpu
from jax.experimental.pallas import tpu_sc as plsc

N = 50000
E = 1600000
F1 = 256
G = 128

R = 4096
NCH = 14
NPAD = R * NCH
BLK = 512
NBLK = NPAD // BLK
NW = 32
ESL = E // NW
TI = 2000
NTI = ESL // TI
B = 128
SEGCAP = 50048
CBCAP = 2272
PADROW = N
SEG_W = NCH * SEGCAP

_mesh = plsc.VectorSubcoreMesh(core_axis_name="c", subcore_axis_name="s")


def _permute(x, idx):
    return lax.gather(
        x, idx[:, None],
        lax.GatherDimensionNumbers(offset_dims=(), collapsed_slice_dims=(0,),
                                   start_index_map=(0,)),
        (1,), mode=lax.GatherScatterMode.PROMISE_IN_BOUNDS)


def _compact16(d16, s16, base, lane):
    m = (d16 >= base) & (d16 < base + R)
    x = jnp.where(m, 1, 0)
    for sh in (1, 2, 4, 8):
        x = x + jnp.where(lane >= sh, _permute(x, jnp.maximum(lane - sh, 0)), 0)
    r16 = lane + 1
    lo = jnp.full((16,), -1, jnp.int32)
    for sh in (8, 4, 2, 1):
        cand = lo + sh
        pv = _permute(x, jnp.minimum(cand, 15))
        lo = jnp.where(pv < r16, cand, lo)
    inv = jnp.minimum(lo + 1, 15)
    return _permute(s16, inv), _permute(d16 - base, inv), x[15]


def _bucket_body(src_hbm, dst_hbm, bsrc_hbm, bdst_hbm, cnt_hbm,
                 srcv, dstv, cev, cov, *cbufs):
    sbk = cbufs[:NCH]
    dbk = cbufs[NCH:2 * NCH]
    c = lax.axis_index("c")
    s = lax.axis_index("s")
    w = s * 2 + c
    lane = lax.broadcasted_iota(jnp.int32, (16,), 0)
    slice_base = w * ESL
    seg0 = w * SEG_W

    def tile_body(ti, carry):
        off = pl.multiple_of(slice_base + ti * TI, 8)
        pltpu.sync_copy(src_hbm.at[pl.ds(off, TI)], srcv)
        pltpu.sync_copy(dst_hbm.at[pl.ds(off, TI)], dstv)
        new = []
        for k in range(NCH):
            ptr, fl = carry[k], carry[NCH + k]
            base = k * R

            def vec_body(i, p, k=k, base=base):
                d16 = dstv[pl.ds(i * 16, 16)]
                s16 = srcv[pl.ds(i * 16, 16)]
                cs, cd, cnt = _compact16(d16, s16, base, lane)
                sbk[k][pl.ds(p, 16)] = cs
                dbk[k][pl.ds(p, 16)] = cd
                return p + cnt

            ptr = lax.fori_loop(0, TI // 16, vec_body, ptr)
            nf = ptr // B

            def flush(f, _, k=k):
                dst_off = pl.multiple_of(seg0 + k * SEGCAP + (fl + f) * B, 8)
                pltpu.sync_copy(sbk[k].at[pl.ds(f * B, B)],
                                bsrc_hbm.at[pl.ds(dst_off, B)])
                pltpu.sync_copy(dbk[k].at[pl.ds(f * B, B)],
                                bdst_hbm.at[pl.ds(dst_off, B)])
                return 0

            lax.fori_loop(0, nf, flush, 0)
            for v in range(B // 16):
                sbk[k][pl.ds(v * 16, 16)] = sbk[k][pl.ds(nf * B + v * 16, 16)]
                dbk[k][pl.ds(v * 16, 16)] = dbk[k][pl.ds(nf * B + v * 16, 16)]
            new.append(ptr - nf * B)
            carry = carry[:NCH + k] + (fl + nf,) + carry[NCH + k + 1:]
        return tuple(new) + carry[NCH:]

    carry = lax.fori_loop(0, NTI, tile_body, (0,) * (2 * NCH))

    cnt_even = jnp.zeros((16,), jnp.int32)
    cnt_odd = jnp.zeros((16,), jnp.int32)
    pad_s = jnp.full((16,), PADROW, jnp.int32)
    pad_d = jnp.zeros((16,), jnp.int32)
    for k in range(NCH):
        ptr, fl = carry[k], carry[NCH + k]
        for v in range(B // 16):
            sbk[k][pl.ds(ptr + v * 16, 16)] = pad_s
            dbk[k][pl.ds(ptr + v * 16, 16)] = pad_d
        dst_off = pl.multiple_of(seg0 + k * SEGCAP + fl * B, 8)
        pltpu.sync_copy(sbk[k].at[pl.ds(0, B)], bsrc_hbm.at[pl.ds(dst_off, B)])
        pltpu.sync_copy(dbk[k].at[pl.ds(0, B)], bdst_hbm.at[pl.ds(dst_off, B)])
        total = fl * B + ptr
        tv = jnp.full((16,), total, jnp.int32)
        sel = jnp.where(lane == (k // 2), tv, 0)
        if k % 2 == 0:
            cnt_even = cnt_even + sel
        else:
            cnt_odd = cnt_odd + sel
    cev[pl.ds(0, 16)] = cnt_even
    cov[pl.ds(0, 16)] = cnt_odd
    pltpu.sync_copy(cev, cnt_hbm.at[pl.ds(pl.multiple_of(w * 16, 8), 16)])
    pltpu.sync_copy(cov, cnt_hbm.at[pl.ds(pl.multiple_of(512 + w * 16, 8), 16)])


_bucket = pl.kernel(
    _bucket_body,
    out_type=(jax.ShapeDtypeStruct((NW * SEG_W,), jnp.int32),
              jax.ShapeDtypeStruct((NW * SEG_W,), jnp.int32),
              jax.ShapeDtypeStruct((1024,), jnp.int32)),
    mesh=_mesh,
    compiler_params=pltpu.CompilerParams(use_tc_tiling_on_sc=False),
    scratch_types=([pltpu.VMEM((TI,), jnp.int32),
                    pltpu.VMEM((TI,), jnp.int32),
                    pltpu.VMEM((16,), jnp.int32),
                    pltpu.VMEM((16,), jnp.int32)]
                   + [pltpu.VMEM((CBCAP,), jnp.int32)
                      for _ in range(2 * NCH)]),
)


def _make_prop(do_gather):
    def body(*refs):
        if do_gather:
            (y_hbm, bsrc_hbm, bdst_hbm, cnt_hbm, out_hbm,
             csrcb, cdstb, gidx, rows, acc, cv, sem) = refs
        else:
            (bsrc_hbm, bdst_hbm, cnt_hbm, out_hbm,
             csrcb, cdstb, gidx, rows, acc, cv, sem) = refs
        c = lax.axis_index("c")
        s = lax.axis_index("s")
        z16 = jnp.zeros((16,), jnp.float32)
        one16 = jnp.ones((16,), jnp.float32)
        pltpu.sync_copy(cnt_hbm.at[pl.ds(pl.multiple_of(c * 512, 8), 512)], cv)

        for j in range(NCH // 2):
            k = 2 * j + c

            def zero_row(i, _):
                acc[i, pl.ds(0, 16)] = z16
                return 0

            lax.fori_loop(0, R, zero_row, 0)

            def prod_body(w, _, j=j):
                cnt = cv[pl.ds(pl.multiple_of(w * 16, 16), 16)][j]
                nbq = (cnt + (B - 1)) // B
                seg = w * SEG_W + k * SEGCAP

                def bat(b, _):
                    boff = pl.multiple_of(seg + b * B, 8)
                    if do_gather:
                        pltpu.sync_copy(bsrc_hbm.at[pl.ds(boff, B)], csrcb)
                        pltpu.sync_copy(bdst_hbm.at[pl.ds(boff, B)], cdstb)
                        for v in range(B // 16):
                            gidx[pl.ds(v * 16, 16)] = (
                                csrcb[pl.ds(v * 16, 16)] + s * NPAD)
                        pltpu.async_copy(y_hbm.at[gidx], rows, sem).wait()
                        for v in range(B // 16):
                            ldv = cdstb[pl.ds(v * 16, 16)]
                            for l in range(16):
                                plsc.addupdate(acc.at[ldv[l]],
                                               rows[v * 16 + l])
                    else:
                        @pl.when((b & 15) == s)
                        def _():
                            pltpu.sync_copy(bdst_hbm.at[pl.ds(boff, B)],
                                            cdstb)
                            for v in range(B // 16):
                                ldv = cdstb[pl.ds(v * 16, 16)]
                                for l in range(16):
                                    plsc.addupdate(acc.at[ldv[l]], one16)
                    return 0

                lax.fori_loop(0, nbq, bat, 0)
                return 0

            lax.fori_loop(0, NW, prod_body, 0)
            pltpu.sync_copy(
                acc, out_hbm.at[pl.ds(pl.multiple_of(s * NPAD + k * R, 8), R)])

    scratch = [
        pltpu.VMEM((B,), jnp.int32),
        pltpu.VMEM((B,), jnp.int32),
        pltpu.VMEM((B,), jnp.int32),
        pltpu.VMEM((B, 16), jnp.float32),
        pltpu.VMEM((R, 16), jnp.float32),
        pltpu.VMEM((512,), jnp.int32),
        pltpu.SemaphoreType.DMA,
    ]
    return pl.kernel(
        body,
        out_type=jax.ShapeDtypeStruct((16 * NPAD, 16), jnp.float32),
        mesh=_mesh,
        scratch_types=scratch,
        compiler_params=pltpu.CompilerParams(use_tc_tiling_on_sc=False),
    )


_prop = _make_prop(True)
_deg_prop = _make_prop(False)


def _dinv_of(deg_ref):
    return lax.rsqrt(deg_ref[:, 0:1] + 1.0)


def _row_mask(i):
    gr = i * BLK + lax.broadcasted_iota(jnp.int32, (BLK, 1), 0)
    return gr < N


def _degsum_body(*refs):
    degs = refs[:16]
    o_ref = refs[16]
    acc = degs[0][...]
    for u in range(1, 16):
        acc = acc + degs[u][...]
    o_ref[...] = acc


_degsum = pl.pallas_call(
    _degsum_body,
    grid=(NBLK,),
    in_specs=[pl.BlockSpec((BLK, 16), lambda i, u=u: (u * NBLK + i, 0))
              for u in range(16)],
    out_specs=pl.BlockSpec((BLK, 16), lambda i: (i, 0)),
    out_shape=jax.ShapeDtypeStruct((NPAD, 16), jnp.float32),
)


_DIMS_NT = (((1,), (1,)), ((), ()))


def _prep1_body(x_ref, wt_ref, deg_ref, o_ref):
    i = pl.program_id(1)
    xw = lax.dot_general(x_ref[...], wt_ref[...], _DIMS_NT,
                         preferred_element_type=jnp.float32)
    y = xw * _dinv_of(deg_ref)
    o_ref[...] = jnp.where(_row_mask(i), y, 0.0)


_prep1 = pl.pallas_call(
    _prep1_body,
    grid=(16, NBLK),
    in_specs=[pl.BlockSpec((BLK, 16), lambda t, i: (i, 0)),
              pl.BlockSpec((16, 16), lambda t, i: (t, 0)),
              pl.BlockSpec((BLK, 16), lambda t, i: (i, 0))],
    out_specs=pl.BlockSpec((BLK, 16), lambda t, i: (t * NBLK + i, 0)),
    out_shape=jax.ShapeDtypeStruct((16 * NPAD, 16), jnp.float32),
)


def _mid_body(*refs):
    accs = refs[:16]
    ys = refs[16:32]
    deg_ref, wt_ref, b_ref, o_ref = refs[32:]
    i = pl.program_id(1)
    dinv = _dinv_of(deg_ref)
    accf = jnp.concatenate([a[...] for a in accs], axis=1)
    yf = jnp.concatenate([y[...] for y in ys], axis=1)
    h = jax.nn.relu(dinv * (accf + yf) + b_ref[0:1, :])
    o = lax.dot_general(h, wt_ref[...], _DIMS_NT,
                        preferred_element_type=jnp.float32) * dinv
    o_ref[...] = jnp.where(_row_mask(i), o, 0.0)


_mid = pl.pallas_call(
    _mid_body,
    grid=(16, NBLK),
    in_specs=([pl.BlockSpec((BLK, 16), lambda t, i, u=u: (u * NBLK + i, 0))
               for u in range(16)]
              + [pl.BlockSpec((BLK, 16), lambda t, i, u=u: (u * NBLK + i, 0))
                 for u in range(16)]
              + [pl.BlockSpec((BLK, 16), lambda t, i: (i, 0)),
                 pl.BlockSpec((16, F1), lambda t, i: (t, 0)),
                 pl.BlockSpec((8, F1), lambda t, i: (0, 0))]),
    out_specs=pl.BlockSpec((BLK, 16), lambda t, i: (t * NBLK + i, 0)),
    out_shape=jax.ShapeDtypeStruct((16 * NPAD, 16), jnp.float32),
)


def _final_body(*refs):
    accs = refs[:16]
    ys = refs[16:32]
    deg_ref, b_ref, batch_ref, sum_ref, cnt_ref = refs[32:]
    i = pl.program_id(0)

    @pl.when(i == 0)
    def _():
        sum_ref[...] = jnp.zeros_like(sum_ref)
        cnt_ref[...] = jnp.zeros_like(cnt_ref)

    dinv = _dinv_of(deg_ref)
    accf = jnp.concatenate([a[...] for a in accs], axis=1)
    yf = jnp.concatenate([y[...] for y in ys], axis=1)
    h = dinv * (accf + yf) + b_ref[0:1, :]
    bb = batch_ref[...].reshape(BLK, 1)
    onehot = (bb == lax.broadcasted_iota(jnp.int32, (BLK, G), 1)).astype(
        jnp.float32)
    dims = (((0,), (0,)), ((), ()))
    sum_ref[...] += lax.dot_general(onehot, h, dims,
                                    preferred_element_type=jnp.float32)
    cnt_ref[...] += lax.dot_general(onehot, jnp.ones((BLK, G), jnp.float32),
                                    dims, preferred_element_type=jnp.float32)

    @pl.when(i == NBLK - 1)
    def _():
        sum_ref[...] = sum_ref[...] / jnp.maximum(cnt_ref[:, 0:1], 1.0)


_final = pl.pallas_call(
    _final_body,
    grid=(NBLK,),
    in_specs=([pl.BlockSpec((BLK, 16), lambda i, u=u: (u * NBLK + i, 0))
               for u in range(16)]
              + [pl.BlockSpec((BLK, 16), lambda i, u=u: (u * NBLK + i, 0))
                 for u in range(16)]
              + [pl.BlockSpec((BLK, 16), lambda i: (i, 0)),
                 pl.BlockSpec((8, F1), lambda i: (0, 0)),
                 pl.BlockSpec((1, 1, BLK), lambda i: (i, 0, 0))]),
    out_specs=[pl.BlockSpec((G, F1), lambda i: (0, 0)),
               pl.BlockSpec((G, G), lambda i: (0, 0))],
    out_shape=[jax.ShapeDtypeStruct((G, F1), jnp.float32),
               jax.ShapeDtypeStruct((G, G), jnp.float32)],
)


def _pad_bias(b):
    return jnp.zeros((8, F1), jnp.float32).at[0].set(b)


def kernel(x, edge_index, batch, W1, b1, W2, b2, W3, b3):
    src = edge_index[0]
    dst = edge_index[1]
    xpad = jnp.zeros((NPAD, 16), jnp.float32).at[:N, :x.shape[1]].set(x)
    w1t = jnp.zeros((F1, 16), jnp.float32).at[:, :W1.shape[0]].set(W1.T)
    w2t = W2.T
    w3t = W3.T
    batch_r = jnp.full((NPAD,), G, jnp.int32).at[:N].set(batch).reshape(
        NBLK, 1, BLK)

    bsrc, bdst, cnts = _bucket(src, dst)
    degp = _deg_prop(bsrc, bdst, cnts)
    degc = _degsum(*([degp] * 16))
    y1 = _prep1(xpad, w1t, degc)
    acc1 = _prop(y1, bsrc, bdst, cnts)
    y2 = _mid(*([acc1] * 16), *([y1] * 16), degc, w2t, _pad_bias(b1))
    acc2 = _prop(y2, bsrc, bdst, cnts)
    y3 = _mid(*([acc2] * 16), *([y2] * 16), degc, w3t, _pad_bias(b2))
    acc3 = _prop(y3, bsrc, bdst, cnts)
    pooled, _ = _final(*([acc3] * 16), *([y3] * 16), degc, _pad_bias(b3),
                       batch_r)
    return pooled


# speedup vs baseline: 1.9526x; 1.4743x over previous
import functools

import jax
import jax.numpy as jnp
from jax import lax
from jax.experimental import pallas as pl
from jax.experimental.pallas import tpu as pltpu
from jax.experimental.pallas import tpu_sc as plsc

N = 50000
E = 1600000
F1 = 256
G = 128

R = 4096
NCH = 14
NPAD = R * NCH
BLK = 512
NBLK = NPAD // BLK
NW = 32
ESL = E // NW
TI = 2000
NTI = ESL // TI
B = 128
SEGCAP = 51200
CBCAP = 2272
PADROW = N
BC = 1024
NVB = BC // 16
SEG_W = NCH * SEGCAP

_mesh = plsc.VectorSubcoreMesh(core_axis_name="c", subcore_axis_name="s")


def _permute(x, idx):
    return lax.gather(
        x, idx[:, None],
        lax.GatherDimensionNumbers(offset_dims=(), collapsed_slice_dims=(0,),
                                   start_index_map=(0,)),
        (1,), mode=lax.GatherScatterMode.PROMISE_IN_BOUNDS)


def _compact16(d16, s16, base, lane):
    m = (d16 >= base) & (d16 < base + R)
    x = jnp.where(m, 1, 0)
    for sh in (1, 2, 4, 8):
        x = x + jnp.where(lane >= sh, _permute(x, jnp.maximum(lane - sh, 0)), 0)
    r16 = lane + 1
    lo = jnp.full((16,), -1, jnp.int32)
    for sh in (8, 4, 2, 1):
        cand = lo + sh
        pv = _permute(x, jnp.minimum(cand, 15))
        lo = jnp.where(pv < r16, cand, lo)
    inv = jnp.minimum(lo + 1, 15)
    return _permute(s16, inv), _permute(d16 - base, inv), x[15]


def _bucket_body(src_hbm, dst_hbm, bsrc_hbm, bdst_hbm, cnt_hbm,
                 srcv, dstv, cev, cov, *cbufs):
    sbk = cbufs[:NCH]
    dbk = cbufs[NCH:2 * NCH]
    c = lax.axis_index("c")
    s = lax.axis_index("s")
    w = s * 2 + c
    lane = lax.broadcasted_iota(jnp.int32, (16,), 0)
    slice_base = w * ESL
    seg0 = w * SEG_W

    def tile_body(ti, carry):
        off = pl.multiple_of(slice_base + ti * TI, 8)
        pltpu.sync_copy(src_hbm.at[pl.ds(off, TI)], srcv)
        pltpu.sync_copy(dst_hbm.at[pl.ds(off, TI)], dstv)
        new = []
        for k in range(NCH):
            ptr, fl = carry[k], carry[NCH + k]
            base = k * R

            def vec_body(i, p, k=k, base=base):
                d16 = dstv[pl.ds(i * 16, 16)]
                s16 = srcv[pl.ds(i * 16, 16)]
                cs, cd, cnt = _compact16(d16, s16, base, lane)
                sbk[k][pl.ds(p, 16)] = cs
                dbk[k][pl.ds(p, 16)] = cd
                return p + cnt

            ptr = lax.fori_loop(0, TI // 16, vec_body, ptr)
            nf = ptr // B

            def flush(f, _, k=k):
                dst_off = pl.multiple_of(seg0 + k * SEGCAP + (fl + f) * B, 8)
                pltpu.sync_copy(sbk[k].at[pl.ds(f * B, B)],
                                bsrc_hbm.at[pl.ds(dst_off, B)])
                pltpu.sync_copy(dbk[k].at[pl.ds(f * B, B)],
                                bdst_hbm.at[pl.ds(dst_off, B)])
                return 0

            lax.fori_loop(0, nf, flush, 0)
            for v in range(B // 16):
                sbk[k][pl.ds(v * 16, 16)] = sbk[k][pl.ds(nf * B + v * 16, 16)]
                dbk[k][pl.ds(v * 16, 16)] = dbk[k][pl.ds(nf * B + v * 16, 16)]
            new.append(ptr - nf * B)
            carry = carry[:NCH + k] + (fl + nf,) + carry[NCH + k + 1:]
        return tuple(new) + carry[NCH:]

    carry = lax.fori_loop(0, NTI, tile_body, (0,) * (2 * NCH))

    cnt_even = jnp.zeros((16,), jnp.int32)
    cnt_odd = jnp.zeros((16,), jnp.int32)
    pad_s = jnp.full((16,), PADROW, jnp.int32)
    pad_d = jnp.zeros((16,), jnp.int32)
    for k in range(NCH):
        ptr, fl = carry[k], carry[NCH + k]
        for v in range(B // 16):
            sbk[k][pl.ds(ptr + v * 16, 16)] = pad_s
            dbk[k][pl.ds(ptr + v * 16, 16)] = pad_d
        dst_off = pl.multiple_of(seg0 + k * SEGCAP + fl * B, 8)
        pltpu.sync_copy(sbk[k].at[pl.ds(0, B)], bsrc_hbm.at[pl.ds(dst_off, B)])
        pltpu.sync_copy(dbk[k].at[pl.ds(0, B)], bdst_hbm.at[pl.ds(dst_off, B)])
        for v in range(B // 16):
            sbk[k][pl.ds(v * 16, 16)] = pad_s
            dbk[k][pl.ds(v * 16, 16)] = pad_d
        nq = fl + 1
        npad_q = (8 - lax.rem(nq, 8)) & 7

        def padflush(f, _, k=k):
            po = pl.multiple_of(seg0 + k * SEGCAP + (nq + f) * B, 8)
            pltpu.sync_copy(sbk[k].at[pl.ds(0, B)], bsrc_hbm.at[pl.ds(po, B)])
            pltpu.sync_copy(dbk[k].at[pl.ds(0, B)], bdst_hbm.at[pl.ds(po, B)])
            return 0

        lax.fori_loop(0, npad_q, padflush, 0)
        total = fl * B + ptr
        tv = jnp.full((16,), total, jnp.int32)
        sel = jnp.where(lane == (k // 2), tv, 0)
        if k % 2 == 0:
            cnt_even = cnt_even + sel
        else:
            cnt_odd = cnt_odd + sel
    cev[pl.ds(0, 16)] = cnt_even
    cov[pl.ds(0, 16)] = cnt_odd
    pltpu.sync_copy(cev, cnt_hbm.at[pl.ds(pl.multiple_of(w * 16, 8), 16)])
    pltpu.sync_copy(cov, cnt_hbm.at[pl.ds(pl.multiple_of(512 + w * 16, 8), 16)])


_bucket = pl.kernel(
    _bucket_body,
    out_type=(jax.ShapeDtypeStruct((NW * SEG_W,), jnp.int32),
              jax.ShapeDtypeStruct((NW * SEG_W,), jnp.int32),
              jax.ShapeDtypeStruct((1024,), jnp.int32)),
    mesh=_mesh,
    compiler_params=pltpu.CompilerParams(use_tc_tiling_on_sc=False),
    scratch_types=([pltpu.VMEM((TI,), jnp.int32),
                    pltpu.VMEM((TI,), jnp.int32),
                    pltpu.VMEM((16,), jnp.int32),
                    pltpu.VMEM((16,), jnp.int32)]
                   + [pltpu.VMEM((CBCAP,), jnp.int32)
                      for _ in range(2 * NCH)]),
)


def _make_prop(do_gather):
    def body(*refs):
        if do_gather:
            (y_hbm, bsrc_hbm, bdst_hbm, cnt_hbm, out_hbm,
             csrcb, cdstb, gidx, rows, acc, cv, sem) = refs
        else:
            (bsrc_hbm, bdst_hbm, cnt_hbm, out_hbm,
             csrcb, cdstb, gidx, rows, acc, cv, sem) = refs
        c = lax.axis_index("c")
        s = lax.axis_index("s")
        z16 = jnp.zeros((16,), jnp.float32)
        one16 = jnp.ones((16,), jnp.float32)
        pltpu.sync_copy(cnt_hbm.at[pl.ds(pl.multiple_of(c * 512, 8), 512)], cv)

        for j in range(NCH // 2):
            k = 2 * j + c

            def zero_row(i, _):
                acc[i, pl.ds(0, 16)] = z16
                return 0

            lax.fori_loop(0, R, zero_row, 0)

            def prod_body(w, _, j=j):
                cnt = cv[pl.ds(pl.multiple_of(w * 16, 16), 16)][j]
                nbq = (cnt + (BC - 1)) // BC
                seg = w * SEG_W + k * SEGCAP

                def bat(b, _):
                    boff = pl.multiple_of(seg + b * BC, 8)
                    if do_gather:
                        pltpu.sync_copy(bsrc_hbm.at[pl.ds(boff, BC)], csrcb)
                        pltpu.sync_copy(bdst_hbm.at[pl.ds(boff, BC)], cdstb)

                        def gix(v, _):
                            gidx[pl.ds(v * 16, 16)] = (
                                csrcb[pl.ds(v * 16, 16)] + s * NPAD)
                            return 0

                        lax.fori_loop(0, NVB, gix, 0)
                        pltpu.async_copy(y_hbm.at[gidx], rows, sem).wait()

                        def accv(v, _):
                            ldv = cdstb[pl.ds(v * 16, 16)]
                            for l in range(16):
                                plsc.addupdate(acc.at[ldv[l]],
                                               rows[v * 16 + l])
                            return 0

                        lax.fori_loop(0, NVB, accv, 0)
                    else:
                        @pl.when((b & 15) == s)
                        def _():
                            pltpu.sync_copy(bdst_hbm.at[pl.ds(boff, BC)],
                                            cdstb)

                            def accv(v, _):
                                ldv = cdstb[pl.ds(v * 16, 16)]
                                for l in range(16):
                                    plsc.addupdate(acc.at[ldv[l]], one16)
                                return 0

                            lax.fori_loop(0, NVB, accv, 0)
                    return 0

                lax.fori_loop(0, nbq, bat, 0)
                return 0

            lax.fori_loop(0, NW, prod_body, 0)
            pltpu.sync_copy(
                acc, out_hbm.at[pl.ds(pl.multiple_of(s * NPAD + k * R, 8), R)])

    scratch = [
        pltpu.VMEM((BC,), jnp.int32),
        pltpu.VMEM((BC,), jnp.int32),
        pltpu.VMEM((BC,), jnp.int32),
        pltpu.VMEM((BC, 16), jnp.float32),
        pltpu.VMEM((R, 16), jnp.float32),
        pltpu.VMEM((512,), jnp.int32),
        pltpu.SemaphoreType.DMA,
    ]
    return pl.kernel(
        body,
        out_type=jax.ShapeDtypeStruct((16 * NPAD, 16), jnp.float32),
        mesh=_mesh,
        scratch_types=scratch,
        compiler_params=pltpu.CompilerParams(use_tc_tiling_on_sc=False),
    )


_prop = _make_prop(True)
_deg_prop = _make_prop(False)


def _dinv_of(deg_ref):
    return lax.rsqrt(deg_ref[:, 0:1] + 1.0)


def _row_mask(i):
    gr = i * BLK + lax.broadcasted_iota(jnp.int32, (BLK, 1), 0)
    return gr < N


def _degsum_body(*refs):
    degs = refs[:16]
    o_ref = refs[16]
    acc = degs[0][...]
    for u in range(1, 16):
        acc = acc + degs[u][...]
    o_ref[...] = acc


_degsum = pl.pallas_call(
    _degsum_body,
    grid=(NBLK,),
    in_specs=[pl.BlockSpec((BLK, 16), lambda i, u=u: (u * NBLK + i, 0))
              for u in range(16)],
    out_specs=pl.BlockSpec((BLK, 16), lambda i: (i, 0)),
    out_shape=jax.ShapeDtypeStruct((NPAD, 16), jnp.float32),
)


_DIMS_NT = (((1,), (1,)), ((), ()))


def _prep1_body(x_ref, wt_ref, deg_ref, o_ref):
    i = pl.program_id(1)
    xw = lax.dot_general(x_ref[...], wt_ref[...], _DIMS_NT,
                         preferred_element_type=jnp.float32)
    y = xw * _dinv_of(deg_ref)
    o_ref[...] = jnp.where(_row_mask(i), y, 0.0)


_prep1 = pl.pallas_call(
    _prep1_body,
    grid=(16, NBLK),
    in_specs=[pl.BlockSpec((BLK, 16), lambda t, i: (i, 0)),
              pl.BlockSpec((16, 16), lambda t, i: (t, 0)),
              pl.BlockSpec((BLK, 16), lambda t, i: (i, 0))],
    out_specs=pl.BlockSpec((BLK, 16), lambda t, i: (t * NBLK + i, 0)),
    out_shape=jax.ShapeDtypeStruct((16 * NPAD, 16), jnp.float32),
)


def _mid_body(*refs):
    accs = refs[:16]
    ys = refs[16:32]
    deg_ref, wt_ref, b_ref, o_ref = refs[32:]
    i = pl.program_id(1)
    dinv = _dinv_of(deg_ref)
    accf = jnp.concatenate([a[...] for a in accs], axis=1)
    yf = jnp.concatenate([y[...] for y in ys], axis=1)
    h = jax.nn.relu(dinv * (accf + yf) + b_ref[0:1, :])
    o = lax.dot_general(h, wt_ref[...], _DIMS_NT,
                        preferred_element_type=jnp.float32) * dinv
    o_ref[...] = jnp.where(_row_mask(i), o, 0.0)


_mid = pl.pallas_call(
    _mid_body,
    grid=(16, NBLK),
    in_specs=([pl.BlockSpec((BLK, 16), lambda t, i, u=u: (u * NBLK + i, 0))
               for u in range(16)]
              + [pl.BlockSpec((BLK, 16), lambda t, i, u=u: (u * NBLK + i, 0))
                 for u in range(16)]
              + [pl.BlockSpec((BLK, 16), lambda t, i: (i, 0)),
                 pl.BlockSpec((16, F1), lambda t, i: (t, 0)),
                 pl.BlockSpec((8, F1), lambda t, i: (0, 0))]),
    out_specs=pl.BlockSpec((BLK, 16), lambda t, i: (t * NBLK + i, 0)),
    out_shape=jax.ShapeDtypeStruct((16 * NPAD, 16), jnp.float32),
)


def _final_body(*refs):
    accs = refs[:16]
    ys = refs[16:32]
    deg_ref, b_ref, batch_ref, sum_ref, cnt_ref = refs[32:]
    i = pl.program_id(0)

    @pl.when(i == 0)
    def _():
        sum_ref[...] = jnp.zeros_like(sum_ref)
        cnt_ref[...] = jnp.zeros_like(cnt_ref)

    dinv = _dinv_of(deg_ref)
    accf = jnp.concatenate([a[...] for a in accs], axis=1)
    yf = jnp.concatenate([y[...] for y in ys], axis=1)
    h = dinv * (accf + yf) + b_ref[0:1, :]
    bb = batch_ref[...].reshape(BLK, 1)
    onehot = (bb == lax.broadcasted_iota(jnp.int32, (BLK, G), 1)).astype(
        jnp.float32)
    dims = (((0,), (0,)), ((), ()))
    sum_ref[...] += lax.dot_general(onehot, h, dims,
                                    preferred_element_type=jnp.float32)
    cnt_ref[...] += lax.dot_general(onehot, jnp.ones((BLK, G), jnp.float32),
                                    dims, preferred_element_type=jnp.float32)

    @pl.when(i == NBLK - 1)
    def _():
        sum_ref[...] = sum_ref[...] / jnp.maximum(cnt_ref[:, 0:1], 1.0)


_final = pl.pallas_call(
    _final_body,
    grid=(NBLK,),
    in_specs=([pl.BlockSpec((BLK, 16), lambda i, u=u: (u * NBLK + i, 0))
               for u in range(16)]
              + [pl.BlockSpec((BLK, 16), lambda i, u=u: (u * NBLK + i, 0))
                 for u in range(16)]
              + [pl.BlockSpec((BLK, 16), lambda i: (i, 0)),
                 pl.BlockSpec((8, F1), lambda i: (0, 0)),
                 pl.BlockSpec((1, 1, BLK), lambda i: (i, 0, 0))]),
    out_specs=[pl.BlockSpec((G, F1), lambda i: (0, 0)),
               pl.BlockSpec((G, G), lambda i: (0, 0))],
    out_shape=[jax.ShapeDtypeStruct((G, F1), jnp.float32),
               jax.ShapeDtypeStruct((G, G), jnp.float32)],
)


def _pad_bias(b):
    return jnp.zeros((8, F1), jnp.float32).at[0].set(b)


def kernel(x, edge_index, batch, W1, b1, W2, b2, W3, b3):
    src = edge_index[0]
    dst = edge_index[1]
    xpad = jnp.zeros((NPAD, 16), jnp.float32).at[:N, :x.shape[1]].set(x)
    w1t = jnp.zeros((F1, 16), jnp.float32).at[:, :W1.shape[0]].set(W1.T)
    w2t = W2.T
    w3t = W3.T
    batch_r = jnp.full((NPAD,), G, jnp.int32).at[:N].set(batch).reshape(
        NBLK, 1, BLK)

    bsrc, bdst, cnts = _bucket(src, dst)
    degp = _deg_prop(bsrc, bdst, cnts)
    degc = _degsum(*([degp] * 16))
    y1 = _prep1(xpad, w1t, degc)
    acc1 = _prop(y1, bsrc, bdst, cnts)
    y2 = _mid(*([acc1] * 16), *([y1] * 16), degc, w2t, _pad_bias(b1))
    acc2 = _prop(y2, bsrc, bdst, cnts)
    y3 = _mid(*([acc2] * 16), *([y2] * 16), degc, w3t, _pad_bias(b2))
    acc3 = _prop(y3, bsrc, bdst, cnts)
    pooled, _ = _final(*([acc3] * 16), *([y3] * 16), degc, _pad_bias(b3),
                       batch_r)
    return pooled


# speedup vs baseline: 2.2960x; 1.1759x over previous
import functools

import jax
import jax.numpy as jnp
from jax import lax
from jax.experimental import pallas as pl
from jax.experimental.pallas import tpu as pltpu
from jax.experimental.pallas import tpu_sc as plsc

N = 50000
E = 1600000
F1 = 256
G = 128

R = 4096
NCH = 14
NPAD = R * NCH
BLK = 512
NBLK = NPAD // BLK
NW = 32
ESL = E // NW
TI = 2000
NTI = ESL // TI
B = 128
SEGCAP = 51200
CBCAP = 2272
PADROW = N
BC = 1024
NVB = BC // 16
SEG_W = NCH * SEGCAP

_mesh = plsc.VectorSubcoreMesh(core_axis_name="c", subcore_axis_name="s")


def _permute(x, idx):
    return lax.gather(
        x, idx[:, None],
        lax.GatherDimensionNumbers(offset_dims=(), collapsed_slice_dims=(0,),
                                   start_index_map=(0,)),
        (1,), mode=lax.GatherScatterMode.PROMISE_IN_BOUNDS)


def _compact16(d16, s16, base, lane):
    m = (d16 >= base) & (d16 < base + R)
    x = jnp.where(m, 1, 0)
    for sh in (1, 2, 4, 8):
        x = x + jnp.where(lane >= sh, _permute(x, jnp.maximum(lane - sh, 0)), 0)
    r16 = lane + 1
    lo = jnp.full((16,), -1, jnp.int32)
    for sh in (8, 4, 2, 1):
        cand = lo + sh
        pv = _permute(x, jnp.minimum(cand, 15))
        lo = jnp.where(pv < r16, cand, lo)
    inv = jnp.minimum(lo + 1, 15)
    return _permute(s16, inv), _permute(d16 - base, inv), x[15]


def _bucket_body(src_hbm, dst_hbm, bsrc_hbm, bdst_hbm, cnt_hbm,
                 srcv, dstv, cev, cov, *cbufs):
    sbk = cbufs[:NCH]
    dbk = cbufs[NCH:2 * NCH]
    c = lax.axis_index("c")
    s = lax.axis_index("s")
    w = s * 2 + c
    lane = lax.broadcasted_iota(jnp.int32, (16,), 0)
    slice_base = w * ESL
    seg0 = w * SEG_W

    def tile_body(ti, carry):
        off = pl.multiple_of(slice_base + ti * TI, 8)
        pltpu.sync_copy(src_hbm.at[pl.ds(off, TI)], srcv)
        pltpu.sync_copy(dst_hbm.at[pl.ds(off, TI)], dstv)
        new = []
        for k in range(NCH):
            ptr, fl = carry[k], carry[NCH + k]
            base = k * R

            def vec_body(i, p, k=k, base=base):
                d16 = dstv[pl.ds(i * 16, 16)]
                s16 = srcv[pl.ds(i * 16, 16)]
                cs, cd, cnt = _compact16(d16, s16, base, lane)
                sbk[k][pl.ds(p, 16)] = cs
                dbk[k][pl.ds(p, 16)] = cd
                return p + cnt

            ptr = lax.fori_loop(0, TI // 16, vec_body, ptr)
            nf = ptr // B

            def flush(f, _, k=k):
                dst_off = pl.multiple_of(seg0 + k * SEGCAP + (fl + f) * B, 8)
                pltpu.sync_copy(sbk[k].at[pl.ds(f * B, B)],
                                bsrc_hbm.at[pl.ds(dst_off, B)])
                pltpu.sync_copy(dbk[k].at[pl.ds(f * B, B)],
                                bdst_hbm.at[pl.ds(dst_off, B)])
                return 0

            lax.fori_loop(0, nf, flush, 0)
            for v in range(B // 16):
                sbk[k][pl.ds(v * 16, 16)] = sbk[k][pl.ds(nf * B + v * 16, 16)]
                dbk[k][pl.ds(v * 16, 16)] = dbk[k][pl.ds(nf * B + v * 16, 16)]
            new.append(ptr - nf * B)
            carry = carry[:NCH + k] + (fl + nf,) + carry[NCH + k + 1:]
        return tuple(new) + carry[NCH:]

    carry = lax.fori_loop(0, NTI, tile_body, (0,) * (2 * NCH))

    cnt_even = jnp.zeros((16,), jnp.int32)
    cnt_odd = jnp.zeros((16,), jnp.int32)
    pad_s = jnp.full((16,), PADROW, jnp.int32)
    pad_d = jnp.zeros((16,), jnp.int32)
    for k in range(NCH):
        ptr, fl = carry[k], carry[NCH + k]
        for v in range(B // 16):
            sbk[k][pl.ds(ptr + v * 16, 16)] = pad_s
            dbk[k][pl.ds(ptr + v * 16, 16)] = pad_d
        dst_off = pl.multiple_of(seg0 + k * SEGCAP + fl * B, 8)
        pltpu.sync_copy(sbk[k].at[pl.ds(0, B)], bsrc_hbm.at[pl.ds(dst_off, B)])
        pltpu.sync_copy(dbk[k].at[pl.ds(0, B)], bdst_hbm.at[pl.ds(dst_off, B)])
        for v in range(B // 16):
            sbk[k][pl.ds(v * 16, 16)] = pad_s
            dbk[k][pl.ds(v * 16, 16)] = pad_d
        nq = fl + 1
        npad_q = (8 - lax.rem(nq, 8)) & 7

        def padflush(f, _, k=k):
            po = pl.multiple_of(seg0 + k * SEGCAP + (nq + f) * B, 8)
            pltpu.sync_copy(sbk[k].at[pl.ds(0, B)], bsrc_hbm.at[pl.ds(po, B)])
            pltpu.sync_copy(dbk[k].at[pl.ds(0, B)], bdst_hbm.at[pl.ds(po, B)])
            return 0

        lax.fori_loop(0, npad_q, padflush, 0)
        total = fl * B + ptr
        tv = jnp.full((16,), total, jnp.int32)
        sel = jnp.where(lane == (k // 2), tv, 0)
        if k % 2 == 0:
            cnt_even = cnt_even + sel
        else:
            cnt_odd = cnt_odd + sel
    cev[pl.ds(0, 16)] = cnt_even
    cov[pl.ds(0, 16)] = cnt_odd
    pltpu.sync_copy(cev, cnt_hbm.at[pl.ds(pl.multiple_of(w * 16, 8), 16)])
    pltpu.sync_copy(cov, cnt_hbm.at[pl.ds(pl.multiple_of(512 + w * 16, 8), 16)])


_bucket = pl.kernel(
    _bucket_body,
    out_type=(jax.ShapeDtypeStruct((NW * SEG_W,), jnp.int32),
              jax.ShapeDtypeStruct((NW * SEG_W,), jnp.int32),
              jax.ShapeDtypeStruct((1024,), jnp.int32)),
    mesh=_mesh,
    compiler_params=pltpu.CompilerParams(use_tc_tiling_on_sc=False),
    scratch_types=([pltpu.VMEM((TI,), jnp.int32),
                    pltpu.VMEM((TI,), jnp.int32),
                    pltpu.VMEM((16,), jnp.int32),
                    pltpu.VMEM((16,), jnp.int32)]
                   + [pltpu.VMEM((CBCAP,), jnp.int32)
                      for _ in range(2 * NCH)]),
)


def _make_prop(do_gather):
    def body(*refs):
        if do_gather:
            (y_hbm, bsrc_hbm, bdst_hbm, cnt_hbm, out_hbm,
             csrcb, cdstb, gidx, rows, csrcb2, cdstb2, gidx2, rows2,
             acc, cv, sem, sem2) = refs
        else:
            (bsrc_hbm, bdst_hbm, cnt_hbm, out_hbm,
             csrcb, cdstb, gidx, rows, csrcb2, cdstb2, gidx2, rows2,
             acc, cv, sem, sem2) = refs
        c = lax.axis_index("c")
        s = lax.axis_index("s")
        z16 = jnp.zeros((16,), jnp.float32)
        one16 = jnp.ones((16,), jnp.float32)
        pltpu.sync_copy(cnt_hbm.at[pl.ds(pl.multiple_of(c * 512, 8), 512)], cv)

        for j in range(NCH // 2):
            k = 2 * j + c

            def zero_row(i, _):
                acc[i, pl.ds(0, 16)] = z16
                return 0

            lax.fori_loop(0, R, zero_row, 0)

            def get_nbq(w):
                cnt = cv[pl.ds(pl.multiple_of(w * 16, 16), 16)][j]
                return jnp.maximum((cnt + (BC - 1)) // BC, 1)

            if do_gather:
                nbtot = lax.fori_loop(0, NW,
                                      lambda w, t: t + get_nbq(w), 0)
                bufs = ((csrcb, cdstb, gidx, rows, sem),
                        (csrcb2, cdstb2, gidx2, rows2, sem2))

                def fire(wf, bf, buf):
                    csb, cdb, gib, rwb, smb = buf
                    boff = pl.multiple_of(wf * SEG_W + k * SEGCAP + bf * BC,
                                          8)
                    pltpu.sync_copy(bsrc_hbm.at[pl.ds(boff, BC)], csb)
                    pltpu.sync_copy(bdst_hbm.at[pl.ds(boff, BC)], cdb)

                    def gix(v, _):
                        gib[pl.ds(v * 16, 16)] = (
                            csb[pl.ds(v * 16, 16)] + s * NPAD)
                        return 0

                    lax.fori_loop(0, NVB, gix, 0)
                    pltpu.async_copy(y_hbm.at[gib], rwb, smb)

                def consume(buf):
                    csb, cdb, gib, rwb, smb = buf
                    pltpu.make_async_copy(y_hbm.at[gib], rwb, smb).wait()

                    def accv(v, _):
                        ldv = cdb[pl.ds(v * 16, 16)]
                        for l in range(16):
                            plsc.addupdate(acc.at[ldv[l]], rwb[v * 16 + l])
                        return 0

                    lax.fori_loop(0, NVB, accv, 0)

                def adv(wf, bf):
                    b2 = bf + 1
                    roll = b2 >= get_nbq(wf)
                    return (jnp.where(roll, wf + 1, wf),
                            jnp.where(roll, 0, b2))

                fire(0, 0, bufs[0])
                st0 = adv(0, 0)

                def pairbody(i, st):
                    wf, bf = st

                    @pl.when(2 * i + 1 < nbtot)
                    def _():
                        fire(wf, bf, bufs[1])

                    wf2, bf2 = adv(wf, bf)
                    cond1 = 2 * i + 1 < nbtot
                    wf = jnp.where(cond1, wf2, wf)
                    bf = jnp.where(cond1, bf2, bf)
                    consume(bufs[0])

                    @pl.when(2 * i + 2 < nbtot)
                    def _():
                        fire(wf, bf, bufs[0])

                    wf2, bf2 = adv(wf, bf)
                    cond2 = 2 * i + 2 < nbtot
                    wf = jnp.where(cond2, wf2, wf)
                    bf = jnp.where(cond2, bf2, bf)

                    @pl.when(cond1)
                    def _():
                        consume(bufs[1])

                    return (wf, bf)

                lax.fori_loop(0, (nbtot + 1) // 2, pairbody, st0)
            else:
                def prod_body(w, _, j=j):
                    nbq = get_nbq(w)
                    seg = w * SEG_W + k * SEGCAP

                    def bat(b, _):
                        boff = pl.multiple_of(seg + b * BC, 8)

                        @pl.when((b & 15) == s)
                        def _():
                            pltpu.sync_copy(bdst_hbm.at[pl.ds(boff, BC)],
                                            cdstb)

                            def accv(v, _):
                                ldv = cdstb[pl.ds(v * 16, 16)]
                                for l in range(16):
                                    plsc.addupdate(acc.at[ldv[l]], one16)
                                return 0

                            lax.fori_loop(0, NVB, accv, 0)
                        return 0

                    lax.fori_loop(0, nbq, bat, 0)
                    return 0

                lax.fori_loop(0, NW, prod_body, 0)
            pltpu.sync_copy(
                acc, out_hbm.at[pl.ds(pl.multiple_of(s * NPAD + k * R, 8), R)])

    scratch = [
        pltpu.VMEM((BC,), jnp.int32),
        pltpu.VMEM((BC,), jnp.int32),
        pltpu.VMEM((BC,), jnp.int32),
        pltpu.VMEM((BC, 16), jnp.float32),
        pltpu.VMEM((BC,), jnp.int32),
        pltpu.VMEM((BC,), jnp.int32),
        pltpu.VMEM((BC,), jnp.int32),
        pltpu.VMEM((BC, 16), jnp.float32),
        pltpu.VMEM((R, 16), jnp.float32),
        pltpu.VMEM((512,), jnp.int32),
        pltpu.SemaphoreType.DMA,
        pltpu.SemaphoreType.DMA,
    ]
    return pl.kernel(
        body,
        out_type=jax.ShapeDtypeStruct((16 * NPAD, 16), jnp.float32),
        mesh=_mesh,
        scratch_types=scratch,
        compiler_params=pltpu.CompilerParams(use_tc_tiling_on_sc=False),
    )


_prop = _make_prop(True)
_deg_prop = _make_prop(False)


def _dinv_of(deg_ref):
    return lax.rsqrt(deg_ref[:, 0:1] + 1.0)


def _row_mask(i):
    gr = i * BLK + lax.broadcasted_iota(jnp.int32, (BLK, 1), 0)
    return gr < N


def _degsum_body(*refs):
    degs = refs[:16]
    o_ref = refs[16]
    acc = degs[0][...]
    for u in range(1, 16):
        acc = acc + degs[u][...]
    o_ref[...] = acc


_degsum = pl.pallas_call(
    _degsum_body,
    grid=(NBLK,),
    in_specs=[pl.BlockSpec((BLK, 16), lambda i, u=u: (u * NBLK + i, 0))
              for u in range(16)],
    out_specs=pl.BlockSpec((BLK, 16), lambda i: (i, 0)),
    out_shape=jax.ShapeDtypeStruct((NPAD, 16), jnp.float32),
)


_DIMS_NT = (((1,), (1,)), ((), ()))


def _prep1_body(x_ref, wt_ref, deg_ref, o_ref):
    i = pl.program_id(1)
    xw = lax.dot_general(x_ref[...], wt_ref[...], _DIMS_NT,
                         preferred_element_type=jnp.float32)
    y = xw * _dinv_of(deg_ref)
    o_ref[...] = jnp.where(_row_mask(i), y, 0.0)


_prep1 = pl.pallas_call(
    _prep1_body,
    grid=(16, NBLK),
    in_specs=[pl.BlockSpec((BLK, 16), lambda t, i: (i, 0)),
              pl.BlockSpec((16, 16), lambda t, i: (t, 0)),
              pl.BlockSpec((BLK, 16), lambda t, i: (i, 0))],
    out_specs=pl.BlockSpec((BLK, 16), lambda t, i: (t * NBLK + i, 0)),
    out_shape=jax.ShapeDtypeStruct((16 * NPAD, 16), jnp.float32),
)


def _mid_body(*refs):
    accs = refs[:16]
    ys = refs[16:32]
    deg_ref, wt_ref, b_ref, o_ref = refs[32:]
    i = pl.program_id(1)
    dinv = _dinv_of(deg_ref)
    accf = jnp.concatenate([a[...] for a in accs], axis=1)
    yf = jnp.concatenate([y[...] for y in ys], axis=1)
    h = jax.nn.relu(dinv * (accf + yf) + b_ref[0:1, :])
    o = lax.dot_general(h, wt_ref[...], _DIMS_NT,
                        preferred_element_type=jnp.float32) * dinv
    o_ref[...] = jnp.where(_row_mask(i), o, 0.0)


_mid = pl.pallas_call(
    _mid_body,
    grid=(16, NBLK),
    in_specs=([pl.BlockSpec((BLK, 16), lambda t, i, u=u: (u * NBLK + i, 0))
               for u in range(16)]
              + [pl.BlockSpec((BLK, 16), lambda t, i, u=u: (u * NBLK + i, 0))
                 for u in range(16)]
              + [pl.BlockSpec((BLK, 16), lambda t, i: (i, 0)),
                 pl.BlockSpec((16, F1), lambda t, i: (t, 0)),
                 pl.BlockSpec((8, F1), lambda t, i: (0, 0))]),
    out_specs=pl.BlockSpec((BLK, 16), lambda t, i: (t * NBLK + i, 0)),
    out_shape=jax.ShapeDtypeStruct((16 * NPAD, 16), jnp.float32),
)


def _final_body(*refs):
    accs = refs[:16]
    ys = refs[16:32]
    deg_ref, b_ref, batch_ref, sum_ref, cnt_ref = refs[32:]
    i = pl.program_id(0)

    @pl.when(i == 0)
    def _():
        sum_ref[...] = jnp.zeros_like(sum_ref)
        cnt_ref[...] = jnp.zeros_like(cnt_ref)

    dinv = _dinv_of(deg_ref)
    accf = jnp.concatenate([a[...] for a in accs], axis=1)
    yf = jnp.concatenate([y[...] for y in ys], axis=1)
    h = dinv * (accf + yf) + b_ref[0:1, :]
    bb = batch_ref[...].reshape(BLK, 1)
    onehot = (bb == lax.broadcasted_iota(jnp.int32, (BLK, G), 1)).astype(
        jnp.float32)
    dims = (((0,), (0,)), ((), ()))
    sum_ref[...] += lax.dot_general(onehot, h, dims,
                                    preferred_element_type=jnp.float32)
    cnt_ref[...] += lax.dot_general(onehot, jnp.ones((BLK, G), jnp.float32),
                                    dims, preferred_element_type=jnp.float32)

    @pl.when(i == NBLK - 1)
    def _():
        sum_ref[...] = sum_ref[...] / jnp.maximum(cnt_ref[:, 0:1], 1.0)


_final = pl.pallas_call(
    _final_body,
    grid=(NBLK,),
    in_specs=([pl.BlockSpec((BLK, 16), lambda i, u=u: (u * NBLK + i, 0))
               for u in range(16)]
              + [pl.BlockSpec((BLK, 16), lambda i, u=u: (u * NBLK + i, 0))
                 for u in range(16)]
              + [pl.BlockSpec((BLK, 16), lambda i: (i, 0)),
                 pl.BlockSpec((8, F1), lambda i: (0, 0)),
                 pl.BlockSpec((1, 1, BLK), lambda i: (i, 0, 0))]),
    out_specs=[pl.BlockSpec((G, F1), lambda i: (0, 0)),
               pl.BlockSpec((G, G), lambda i: (0, 0))],
    out_shape=[jax.ShapeDtypeStruct((G, F1), jnp.float32),
               jax.ShapeDtypeStruct((G, G), jnp.float32)],
)


def _pad_bias(b):
    return jnp.zeros((8, F1), jnp.float32).at[0].set(b)


def kernel(x, edge_index, batch, W1, b1, W2, b2, W3, b3):
    src = edge_index[0]
    dst = edge_index[1]
    xpad = jnp.zeros((NPAD, 16), jnp.float32).at[:N, :x.shape[1]].set(x)
    w1t = jnp.zeros((F1, 16), jnp.float32).at[:, :W1.shape[0]].set(W1.T)
    w2t = W2.T
    w3t = W3.T
    batch_r = jnp.full((NPAD,), G, jnp.int32).at[:N].set(batch).reshape(
        NBLK, 1, BLK)

    bsrc, bdst, cnts = _bucket(src, dst)
    degp = _deg_prop(bsrc, bdst, cnts)
    degc = _degsum(*([degp] * 16))
    y1 = _prep1(xpad, w1t, degc)
    acc1 = _prop(y1, bsrc, bdst, cnts)
    y2 = _mid(*([acc1] * 16), *([y1] * 16), degc, w2t, _pad_bias(b1))
    acc2 = _prop(y2, bsrc, bdst, cnts)
    y3 = _mid(*([acc2] * 16), *([y2] * 16), degc, w3t, _pad_bias(b2))
    acc3 = _prop(y3, bsrc, bdst, cnts)
    pooled, _ = _final(*([acc3] * 16), *([y3] * 16), degc, _pad_bias(b3),
                       batch_r)
    return pooled


# speedup vs baseline: 2.9365x; 1.2790x over previous
import functools

import jax
import jax.numpy as jnp
from jax import lax
from jax.experimental import pallas as pl
from jax.experimental.pallas import tpu as pltpu
from jax.experimental.pallas import tpu_sc as plsc

N = 50000
E = 1600000
F1 = 256
G = 128

R = 4096
NCH = 14
NPAD = R * NCH
BLK = 512
NBLK = NPAD // BLK
NW = 32
ESL = E // NW
TI = 2000
NTI = ESL // TI
B = 128
SEGCAP = 51200
CBCAP = 2272
PADROW = N
BC = 1024
NVB = BC // 16
SEG_W = NCH * SEGCAP

_mesh = plsc.VectorSubcoreMesh(core_axis_name="c", subcore_axis_name="s")


def _permute(x, idx):
    return lax.gather(
        x, idx[:, None],
        lax.GatherDimensionNumbers(offset_dims=(), collapsed_slice_dims=(0,),
                                   start_index_map=(0,)),
        (1,), mode=lax.GatherScatterMode.PROMISE_IN_BOUNDS)


def _compact16(d16, s16, base, lane):
    m = (d16 >= base) & (d16 < base + R)
    x = jnp.where(m, 1, 0)
    for sh in (1, 2, 4, 8):
        x = x + jnp.where(lane >= sh, _permute(x, jnp.maximum(lane - sh, 0)), 0)
    r16 = lane + 1
    lo = jnp.full((16,), -1, jnp.int32)
    for sh in (8, 4, 2, 1):
        cand = lo + sh
        pv = _permute(x, jnp.minimum(cand, 15))
        lo = jnp.where(pv < r16, cand, lo)
    inv = jnp.minimum(lo + 1, 15)
    return _permute(s16, inv), _permute(d16 - base, inv), x[15]


def _bucket_body(src_hbm, dst_hbm, bsrc_hbm, bdst_hbm, cnt_hbm,
                 srcv, dstv, cev, cov, *cbufs):
    sbk = cbufs[:NCH]
    dbk = cbufs[NCH:2 * NCH]
    c = lax.axis_index("c")
    s = lax.axis_index("s")
    w = s * 2 + c
    lane = lax.broadcasted_iota(jnp.int32, (16,), 0)
    slice_base = w * ESL
    seg0 = w * SEG_W

    def tile_body(ti, carry):
        off = pl.multiple_of(slice_base + ti * TI, 8)
        pltpu.sync_copy(src_hbm.at[pl.ds(off, TI)], srcv)
        pltpu.sync_copy(dst_hbm.at[pl.ds(off, TI)], dstv)
        new = []
        for k in range(NCH):
            ptr, fl = carry[k], carry[NCH + k]
            base = k * R

            def vec_body(i, p, k=k, base=base):
                d16 = dstv[pl.ds(i * 16, 16)]
                s16 = srcv[pl.ds(i * 16, 16)]
                cs, cd, cnt = _compact16(d16, s16, base, lane)
                sbk[k][pl.ds(p, 16)] = cs
                dbk[k][pl.ds(p, 16)] = cd
                return p + cnt

            ptr = lax.fori_loop(0, TI // 16, vec_body, ptr)
            nf = ptr // B

            def flush(f, _, k=k):
                dst_off = pl.multiple_of(seg0 + k * SEGCAP + (fl + f) * B, 8)
                pltpu.sync_copy(sbk[k].at[pl.ds(f * B, B)],
                                bsrc_hbm.at[pl.ds(dst_off, B)])
                pltpu.sync_copy(dbk[k].at[pl.ds(f * B, B)],
                                bdst_hbm.at[pl.ds(dst_off, B)])
                return 0

            lax.fori_loop(0, nf, flush, 0)
            for v in range(B // 16):
                sbk[k][pl.ds(v * 16, 16)] = sbk[k][pl.ds(nf * B + v * 16, 16)]
                dbk[k][pl.ds(v * 16, 16)] = dbk[k][pl.ds(nf * B + v * 16, 16)]
            new.append(ptr - nf * B)
            carry = carry[:NCH + k] + (fl + nf,) + carry[NCH + k + 1:]
        return tuple(new) + carry[NCH:]

    carry = lax.fori_loop(0, NTI, tile_body, (0,) * (2 * NCH))

    cnt_even = jnp.zeros((16,), jnp.int32)
    cnt_odd = jnp.zeros((16,), jnp.int32)
    pad_s = jnp.full((16,), PADROW, jnp.int32)
    pad_d = jnp.zeros((16,), jnp.int32)
    for k in range(NCH):
        ptr, fl = carry[k], carry[NCH + k]
        for v in range(B // 16):
            sbk[k][pl.ds(ptr + v * 16, 16)] = pad_s
            dbk[k][pl.ds(ptr + v * 16, 16)] = pad_d
        dst_off = pl.multiple_of(seg0 + k * SEGCAP + fl * B, 8)
        pltpu.sync_copy(sbk[k].at[pl.ds(0, B)], bsrc_hbm.at[pl.ds(dst_off, B)])
        pltpu.sync_copy(dbk[k].at[pl.ds(0, B)], bdst_hbm.at[pl.ds(dst_off, B)])
        for v in range(B // 16):
            sbk[k][pl.ds(v * 16, 16)] = pad_s
            dbk[k][pl.ds(v * 16, 16)] = pad_d
        nq = fl + 1
        npad_q = (8 - lax.rem(nq, 8)) & 7

        def padflush(f, _, k=k):
            po = pl.multiple_of(seg0 + k * SEGCAP + (nq + f) * B, 8)
            pltpu.sync_copy(sbk[k].at[pl.ds(0, B)], bsrc_hbm.at[pl.ds(po, B)])
            pltpu.sync_copy(dbk[k].at[pl.ds(0, B)], bdst_hbm.at[pl.ds(po, B)])
            return 0

        lax.fori_loop(0, npad_q, padflush, 0)
        total = fl * B + ptr
        tv = jnp.full((16,), total, jnp.int32)
        sel = jnp.where(lane == (k // 2), tv, 0)
        if k % 2 == 0:
            cnt_even = cnt_even + sel
        else:
            cnt_odd = cnt_odd + sel
    cev[pl.ds(0, 16)] = cnt_even
    cov[pl.ds(0, 16)] = cnt_odd
    pltpu.sync_copy(cev, cnt_hbm.at[pl.ds(pl.multiple_of(w * 16, 8), 16)])
    pltpu.sync_copy(cov, cnt_hbm.at[pl.ds(pl.multiple_of(512 + w * 16, 8), 16)])


_bucket = pl.kernel(
    _bucket_body,
    out_type=(jax.ShapeDtypeStruct((NW * SEG_W,), jnp.int32),
              jax.ShapeDtypeStruct((NW * SEG_W,), jnp.int32),
              jax.ShapeDtypeStruct((1024,), jnp.int32)),
    mesh=_mesh,
    compiler_params=pltpu.CompilerParams(use_tc_tiling_on_sc=False),
    scratch_types=([pltpu.VMEM((TI,), jnp.int32),
                    pltpu.VMEM((TI,), jnp.int32),
                    pltpu.VMEM((16,), jnp.int32),
                    pltpu.VMEM((16,), jnp.int32)]
                   + [pltpu.VMEM((CBCAP,), jnp.int32)
                      for _ in range(2 * NCH)]),
)


def _make_prop(do_gather):
    def body(*refs):
        if do_gather:
            (y_hbm, bsrc_hbm, bdst_hbm, cnt_hbm, out_hbm,
             csrcb, cdstb, gidx, rows, csrcb2, cdstb2, gidx2, rows2,
             acc, cv, sem, sem2) = refs
        else:
            (bsrc_hbm, bdst_hbm, cnt_hbm, out_hbm,
             csrcb, cdstb, gidx, rows, csrcb2, cdstb2, gidx2, rows2,
             acc, cv, sem, sem2) = refs
        c = lax.axis_index("c")
        s = lax.axis_index("s")
        z16 = jnp.zeros((16,), jnp.float32)
        one16 = jnp.ones((16,), jnp.float32)
        pltpu.sync_copy(cnt_hbm.at[pl.ds(pl.multiple_of(c * 512, 8), 512)], cv)

        for j in range(NCH // 2):
            k = 2 * j + c

            def zero_row(i, _):
                acc[i, pl.ds(0, 16)] = z16
                return 0

            lax.fori_loop(0, R, zero_row, 0)

            def get_nbq(w):
                cnt = cv[pl.ds(pl.multiple_of(w * 16, 16), 16)][j]
                return jnp.maximum((cnt + (BC - 1)) // BC, 1)

            if do_gather:
                nbtot = lax.fori_loop(0, NW,
                                      lambda w, t: t + get_nbq(w), 0)
                bufs = ((csrcb, cdstb, gidx, rows, sem),
                        (csrcb2, cdstb2, gidx2, rows2, sem2))

                def fire(wf, bf, buf):
                    csb, cdb, gib, rwb, smb = buf
                    boff = pl.multiple_of(wf * SEG_W + k * SEGCAP + bf * BC,
                                          8)
                    pltpu.sync_copy(bsrc_hbm.at[pl.ds(boff, BC)], csb)
                    pltpu.sync_copy(bdst_hbm.at[pl.ds(boff, BC)], cdb)

                    def gix(v, _):
                        gib[pl.ds(v * 16, 16)] = (
                            csb[pl.ds(v * 16, 16)] + s * NPAD)
                        return 0

                    lax.fori_loop(0, NVB, gix, 0)
                    pltpu.async_copy(y_hbm.at[gib], rwb, smb)

                def consume(buf):
                    csb, cdb, gib, rwb, smb = buf
                    pltpu.make_async_copy(y_hbm.at[gib], rwb, smb).wait()

                    def accv(v, _):
                        ldv = cdb[pl.ds(v * 16, 16)]
                        for l in range(16):
                            plsc.addupdate(acc.at[ldv[l]], rwb[v * 16 + l])
                        return 0

                    lax.fori_loop(0, NVB, accv, 0)

                def adv(wf, bf):
                    b2 = bf + 1
                    roll = b2 >= get_nbq(wf)
                    return (jnp.where(roll, wf + 1, wf),
                            jnp.where(roll, 0, b2))

                fire(0, 0, bufs[0])
                st0 = adv(0, 0)

                def pairbody(i, st):
                    wf, bf = st

                    @pl.when(2 * i + 1 < nbtot)
                    def _():
                        fire(wf, bf, bufs[1])

                    wf2, bf2 = adv(wf, bf)
                    cond1 = 2 * i + 1 < nbtot
                    wf = jnp.where(cond1, wf2, wf)
                    bf = jnp.where(cond1, bf2, bf)
                    consume(bufs[0])

                    @pl.when(2 * i + 2 < nbtot)
                    def _():
                        fire(wf, bf, bufs[0])

                    wf2, bf2 = adv(wf, bf)
                    cond2 = 2 * i + 2 < nbtot
                    wf = jnp.where(cond2, wf2, wf)
                    bf = jnp.where(cond2, bf2, bf)

                    @pl.when(cond1)
                    def _():
                        consume(bufs[1])

                    return (wf, bf)

                lax.fori_loop(0, (nbtot + 1) // 2, pairbody, st0)
            else:
                def prod_body(w, _, j=j):
                    nbq = get_nbq(w)
                    seg = w * SEG_W + k * SEGCAP

                    def bat(b, _):
                        boff = pl.multiple_of(seg + b * BC, 8)

                        @pl.when((b & 15) == s)
                        def _():
                            pltpu.sync_copy(bdst_hbm.at[pl.ds(boff, BC)],
                                            cdstb)

                            def accv(v, _):
                                ldv = cdstb[pl.ds(v * 16, 16)]
                                for l in range(16):
                                    plsc.addupdate(acc.at[ldv[l]], one16)
                                return 0

                            lax.fori_loop(0, NVB, accv, 0)
                        return 0

                    lax.fori_loop(0, nbq, bat, 0)
                    return 0

                lax.fori_loop(0, NW, prod_body, 0)
            pltpu.sync_copy(
                acc, out_hbm.at[pl.ds(pl.multiple_of(s * NPAD + k * R, 8), R)])

    scratch = [
        pltpu.VMEM((BC,), jnp.int32),
        pltpu.VMEM((BC,), jnp.int32),
        pltpu.VMEM((BC,), jnp.int32),
        pltpu.VMEM((BC, 16), jnp.float32),
        pltpu.VMEM((BC,), jnp.int32),
        pltpu.VMEM((BC,), jnp.int32),
        pltpu.VMEM((BC,), jnp.int32),
        pltpu.VMEM((BC, 16), jnp.float32),
        pltpu.VMEM((R, 16), jnp.float32),
        pltpu.VMEM((512,), jnp.int32),
        pltpu.SemaphoreType.DMA,
        pltpu.SemaphoreType.DMA,
    ]
    return pl.kernel(
        body,
        out_type=jax.ShapeDtypeStruct((16 * NPAD, 16), jnp.float32),
        mesh=_mesh,
        scratch_types=scratch,
        compiler_params=pltpu.CompilerParams(use_tc_tiling_on_sc=False),
    )


_prop = _make_prop(True)
_deg_prop = _make_prop(False)


def _dinv_of(deg_ref):
    return lax.rsqrt(deg_ref[:, 0:1] + 1.0)


def _row_mask(i):
    gr = i * BLK + lax.broadcasted_iota(jnp.int32, (BLK, 1), 0)
    return gr < N


def _degsum_body(*refs):
    degs = refs[:16]
    o_ref = refs[16]
    acc = degs[0][...]
    for u in range(1, 16):
        acc = acc + degs[u][...]
    o_ref[...] = acc


_degsum = pl.pallas_call(
    _degsum_body,
    grid=(NBLK,),
    in_specs=[pl.BlockSpec((BLK, 16), lambda i, u=u: (u * NBLK + i, 0))
              for u in range(16)],
    out_specs=pl.BlockSpec((BLK, 16), lambda i: (i, 0)),
    out_shape=jax.ShapeDtypeStruct((NPAD, 16), jnp.float32),
)


_DIMS_NT = (((1,), (1,)), ((), ()))


def _prep1_body(x_ref, wt_ref, deg_ref, o_ref):
    i = pl.program_id(1)
    xw = lax.dot_general(x_ref[...], wt_ref[...], _DIMS_NT,
                         preferred_element_type=jnp.float32)
    y = xw * _dinv_of(deg_ref)
    o_ref[...] = jnp.where(_row_mask(i), y, 0.0)


_prep1 = pl.pallas_call(
    _prep1_body,
    grid=(16, NBLK),
    in_specs=[pl.BlockSpec((BLK, 16), lambda t, i: (i, 0)),
              pl.BlockSpec((16, 16), lambda t, i: (t, 0)),
              pl.BlockSpec((BLK, 16), lambda t, i: (i, 0))],
    out_specs=pl.BlockSpec((BLK, 16), lambda t, i: (t * NBLK + i, 0)),
    out_shape=jax.ShapeDtypeStruct((16 * NPAD, 16), jnp.float32),
)


def _h_body(*refs):
    accs = refs[:16]
    ys = refs[16:32]
    deg_ref, b_ref, o_ref = refs[32:]
    dinv = _dinv_of(deg_ref)
    accf = jnp.concatenate([a[...] for a in accs], axis=1)
    yf = jnp.concatenate([y[...] for y in ys], axis=1)
    o_ref[...] = jax.nn.relu(dinv * (accf + yf) + b_ref[0:1, :])


_hk = pl.pallas_call(
    _h_body,
    grid=(NBLK,),
    in_specs=([pl.BlockSpec((BLK, 16), lambda i, u=u: (u * NBLK + i, 0))
               for u in range(16)]
              + [pl.BlockSpec((BLK, 16), lambda i, u=u: (u * NBLK + i, 0))
                 for u in range(16)]
              + [pl.BlockSpec((BLK, 16), lambda i: (i, 0)),
                 pl.BlockSpec((8, F1), lambda i: (0, 0))]),
    out_specs=pl.BlockSpec((BLK, F1), lambda i: (i, 0)),
    out_shape=jax.ShapeDtypeStruct((NPAD, F1), jnp.float32),
)


def _y_body(h_ref, deg_ref, wt_ref, o_ref):
    i = pl.program_id(1)
    o = lax.dot_general(h_ref[...], wt_ref[...], _DIMS_NT,
                        preferred_element_type=jnp.float32) * _dinv_of(deg_ref)
    o_ref[...] = jnp.where(_row_mask(i), o, 0.0)


_yk = pl.pallas_call(
    _y_body,
    grid=(16, NBLK),
    in_specs=[pl.BlockSpec((BLK, F1), lambda t, i: (i, 0)),
              pl.BlockSpec((BLK, 16), lambda t, i: (i, 0)),
              pl.BlockSpec((16, F1), lambda t, i: (t, 0))],
    out_specs=pl.BlockSpec((BLK, 16), lambda t, i: (t * NBLK + i, 0)),
    out_shape=jax.ShapeDtypeStruct((16 * NPAD, 16), jnp.float32),
)


def _mid(*args):
    accs_ys = args[:32]
    degc, wt, bp = args[32:]
    h = _hk(*accs_ys, degc, bp)
    return _yk(h, degc, wt)


def _final_body(*refs):
    accs = refs[:16]
    ys = refs[16:32]
    deg_ref, b_ref, batch_ref, sum_ref, cnt_ref = refs[32:]
    i = pl.program_id(0)

    @pl.when(i == 0)
    def _():
        sum_ref[...] = jnp.zeros_like(sum_ref)
        cnt_ref[...] = jnp.zeros_like(cnt_ref)

    dinv = _dinv_of(deg_ref)
    accf = jnp.concatenate([a[...] for a in accs], axis=1)
    yf = jnp.concatenate([y[...] for y in ys], axis=1)
    h = dinv * (accf + yf) + b_ref[0:1, :]
    bb = batch_ref[...].reshape(BLK, 1)
    onehot = (bb == lax.broadcasted_iota(jnp.int32, (BLK, G), 1)).astype(
        jnp.float32)
    dims = (((0,), (0,)), ((), ()))
    sum_ref[...] += lax.dot_general(onehot, h, dims,
                                    preferred_element_type=jnp.float32)
    cnt_ref[...] += lax.dot_general(onehot, jnp.ones((BLK, G), jnp.float32),
                                    dims, preferred_element_type=jnp.float32)

    @pl.when(i == NBLK - 1)
    def _():
        sum_ref[...] = sum_ref[...] / jnp.maximum(cnt_ref[:, 0:1], 1.0)


_final = pl.pallas_call(
    _final_body,
    grid=(NBLK,),
    in_specs=([pl.BlockSpec((BLK, 16), lambda i, u=u: (u * NBLK + i, 0))
               for u in range(16)]
              + [pl.BlockSpec((BLK, 16), lambda i, u=u: (u * NBLK + i, 0))
                 for u in range(16)]
              + [pl.BlockSpec((BLK, 16), lambda i: (i, 0)),
                 pl.BlockSpec((8, F1), lambda i: (0, 0)),
                 pl.BlockSpec((1, 1, BLK), lambda i: (i, 0, 0))]),
    out_specs=[pl.BlockSpec((G, F1), lambda i: (0, 0)),
               pl.BlockSpec((G, G), lambda i: (0, 0))],
    out_shape=[jax.ShapeDtypeStruct((G, F1), jnp.float32),
               jax.ShapeDtypeStruct((G, G), jnp.float32)],
)


def _pad_bias(b):
    return jnp.zeros((8, F1), jnp.float32).at[0].set(b)


def kernel(x, edge_index, batch, W1, b1, W2, b2, W3, b3):
    src = edge_index[0]
    dst = edge_index[1]
    xpad = jnp.zeros((NPAD, 16), jnp.float32).at[:N, :x.shape[1]].set(x)
    w1t = jnp.zeros((F1, 16), jnp.float32).at[:, :W1.shape[0]].set(W1.T)
    w2t = W2.T
    w3t = W3.T
    batch_r = jnp.full((NPAD,), G, jnp.int32).at[:N].set(batch).reshape(
        NBLK, 1, BLK)

    bsrc, bdst, cnts = _bucket(src, dst)
    degp = _deg_prop(bsrc, bdst, cnts)
    degc = _degsum(*([degp] * 16))
    y1 = _prep1(xpad, w1t, degc)
    acc1 = _prop(y1, bsrc, bdst, cnts)
    y2 = _mid(*([acc1] * 16), *([y1] * 16), degc, w2t, _pad_bias(b1))
    acc2 = _prop(y2, bsrc, bdst, cnts)
    y3 = _mid(*([acc2] * 16), *([y2] * 16), degc, w3t, _pad_bias(b2))
    acc3 = _prop(y3, bsrc, bdst, cnts)
    pooled, _ = _final(*([acc3] * 16), *([y3] * 16), degc, _pad_bias(b3),
                       batch_r)
    return pooled


# speedup vs baseline: 3.1929x; 1.0873x over previous
import functools

import jax
import jax.numpy as jnp
from jax import lax
from jax.experimental import pallas as pl
from jax.experimental.pallas import tpu as pltpu
from jax.experimental.pallas import tpu_sc as plsc

N = 50000
E = 1600000
F1 = 256
G = 128

R = 4096
NCH = 14
NPAD = R * NCH
BLK = 512
NBLK = NPAD // BLK
NW = 32
ESL = E // NW
TI = 2000
NTI = ESL // TI
B = 128
SEGCAP = 51200
CBCAP = 2272
PADROW = N
BC = 1024
NVB = BC // 16
SEG_W = NCH * SEGCAP

_mesh = plsc.VectorSubcoreMesh(core_axis_name="c", subcore_axis_name="s")


def _permute(x, idx):
    return lax.gather(
        x, idx[:, None],
        lax.GatherDimensionNumbers(offset_dims=(), collapsed_slice_dims=(0,),
                                   start_index_map=(0,)),
        (1,), mode=lax.GatherScatterMode.PROMISE_IN_BOUNDS)


def _compact16(d16, s16, base, lane):
    m = (d16 >= base) & (d16 < base + R)
    x = jnp.where(m, 1, 0)
    for sh in (1, 2, 4, 8):
        x = x + jnp.where(lane >= sh, _permute(x, jnp.maximum(lane - sh, 0)), 0)
    r16 = lane + 1
    lo = jnp.full((16,), -1, jnp.int32)
    for sh in (8, 4, 2, 1):
        cand = lo + sh
        pv = _permute(x, jnp.minimum(cand, 15))
        lo = jnp.where(pv < r16, cand, lo)
    inv = jnp.minimum(lo + 1, 15)
    return _permute(s16, inv), _permute(d16 - base, inv), x[15]


def _bucket_body(src_hbm, dst_hbm, bsrc_hbm, bdst_hbm, cnt_hbm,
                 srcv, dstv, cev, cov, *cbufs):
    sbk = cbufs[:NCH]
    dbk = cbufs[NCH:2 * NCH]
    c = lax.axis_index("c")
    s = lax.axis_index("s")
    w = s * 2 + c
    lane = lax.broadcasted_iota(jnp.int32, (16,), 0)
    slice_base = w * ESL
    seg0 = w * SEG_W

    def tile_body(ti, carry):
        off = pl.multiple_of(slice_base + ti * TI, 8)
        pltpu.sync_copy(src_hbm.at[pl.ds(off, TI)], srcv)
        pltpu.sync_copy(dst_hbm.at[pl.ds(off, TI)], dstv)
        new = []
        for k in range(NCH):
            ptr, fl = carry[k], carry[NCH + k]
            base = k * R

            def vec_body(i, p, k=k, base=base):
                d16 = dstv[pl.ds(i * 16, 16)]
                s16 = srcv[pl.ds(i * 16, 16)]
                cs, cd, cnt = _compact16(d16, s16, base, lane)
                sbk[k][pl.ds(p, 16)] = cs
                dbk[k][pl.ds(p, 16)] = cd
                return p + cnt

            ptr = lax.fori_loop(0, TI // 16, vec_body, ptr)
            nf = ptr // B

            def flush(f, _, k=k):
                dst_off = pl.multiple_of(seg0 + k * SEGCAP + (fl + f) * B, 8)
                pltpu.sync_copy(sbk[k].at[pl.ds(f * B, B)],
                                bsrc_hbm.at[pl.ds(dst_off, B)])
                pltpu.sync_copy(dbk[k].at[pl.ds(f * B, B)],
                                bdst_hbm.at[pl.ds(dst_off, B)])
                return 0

            lax.fori_loop(0, nf, flush, 0)
            for v in range(B // 16):
                sbk[k][pl.ds(v * 16, 16)] = sbk[k][pl.ds(nf * B + v * 16, 16)]
                dbk[k][pl.ds(v * 16, 16)] = dbk[k][pl.ds(nf * B + v * 16, 16)]
            new.append(ptr - nf * B)
            carry = carry[:NCH + k] + (fl + nf,) + carry[NCH + k + 1:]
        return tuple(new) + carry[NCH:]

    carry = lax.fori_loop(0, NTI, tile_body, (0,) * (2 * NCH))

    cnt_even = jnp.zeros((16,), jnp.int32)
    cnt_odd = jnp.zeros((16,), jnp.int32)
    pad_s = jnp.full((16,), PADROW, jnp.int32)
    pad_d = jnp.zeros((16,), jnp.int32)
    for k in range(NCH):
        ptr, fl = carry[k], carry[NCH + k]
        for v in range(B // 16):
            sbk[k][pl.ds(ptr + v * 16, 16)] = pad_s
            dbk[k][pl.ds(ptr + v * 16, 16)] = pad_d
        dst_off = pl.multiple_of(seg0 + k * SEGCAP + fl * B, 8)
        pltpu.sync_copy(sbk[k].at[pl.ds(0, B)], bsrc_hbm.at[pl.ds(dst_off, B)])
        pltpu.sync_copy(dbk[k].at[pl.ds(0, B)], bdst_hbm.at[pl.ds(dst_off, B)])
        for v in range(B // 16):
            sbk[k][pl.ds(v * 16, 16)] = pad_s
            dbk[k][pl.ds(v * 16, 16)] = pad_d
        nq = fl + 1
        npad_q = (8 - lax.rem(nq, 8)) & 7

        def padflush(f, _, k=k):
            po = pl.multiple_of(seg0 + k * SEGCAP + (nq + f) * B, 8)
            pltpu.sync_copy(sbk[k].at[pl.ds(0, B)], bsrc_hbm.at[pl.ds(po, B)])
            pltpu.sync_copy(dbk[k].at[pl.ds(0, B)], bdst_hbm.at[pl.ds(po, B)])
            return 0

        lax.fori_loop(0, npad_q, padflush, 0)
        total = fl * B + ptr
        tv = jnp.full((16,), total, jnp.int32)
        sel = jnp.where(lane == (k // 2), tv, 0)
        if k % 2 == 0:
            cnt_even = cnt_even + sel
        else:
            cnt_odd = cnt_odd + sel
    cev[pl.ds(0, 16)] = cnt_even
    cov[pl.ds(0, 16)] = cnt_odd
    pltpu.sync_copy(cev, cnt_hbm.at[pl.ds(pl.multiple_of(w * 16, 8), 16)])
    pltpu.sync_copy(cov, cnt_hbm.at[pl.ds(pl.multiple_of(512 + w * 16, 8), 16)])


_bucket = pl.kernel(
    _bucket_body,
    out_type=(jax.ShapeDtypeStruct((NW * SEG_W,), jnp.int32),
              jax.ShapeDtypeStruct((NW * SEG_W,), jnp.int32),
              jax.ShapeDtypeStruct((1024,), jnp.int32)),
    mesh=_mesh,
    compiler_params=pltpu.CompilerParams(use_tc_tiling_on_sc=False),
    scratch_types=([pltpu.VMEM((TI,), jnp.int32),
                    pltpu.VMEM((TI,), jnp.int32),
                    pltpu.VMEM((16,), jnp.int32),
                    pltpu.VMEM((16,), jnp.int32)]
                   + [pltpu.VMEM((CBCAP,), jnp.int32)
                      for _ in range(2 * NCH)]),
)


def _make_prop(do_gather):
    def body(*refs):
        if do_gather:
            (y_hbm, bsrc_hbm, bdst_hbm, cnt_hbm, out_hbm,
             csrcb, cdstb, gidx, rows, csrcb2, cdstb2, gidx2, rows2,
             acc, cv, sem, sem2) = refs
        else:
            (bsrc_hbm, bdst_hbm, cnt_hbm, out_hbm,
             csrcb, cdstb, gidx, rows, csrcb2, cdstb2, gidx2, rows2,
             acc, cv, sem, sem2) = refs
        c = lax.axis_index("c")
        s = lax.axis_index("s")
        z16 = jnp.zeros((16,), jnp.float32)
        one16 = jnp.ones((16,), jnp.float32)
        pltpu.sync_copy(cnt_hbm.at[pl.ds(pl.multiple_of(c * 512, 8), 512)], cv)

        for j in range(NCH // 2):
            k = 2 * j + c

            def zero_row(i, _):
                acc[i, pl.ds(0, 16)] = z16
                return 0

            lax.fori_loop(0, R, zero_row, 0)

            def get_nbq(w):
                cnt = cv[pl.ds(pl.multiple_of(w * 16, 16), 16)][j]
                return jnp.maximum((cnt + (BC - 1)) // BC, 1)

            if do_gather:
                nbtot = lax.fori_loop(0, NW,
                                      lambda w, t: t + get_nbq(w), 0)
                bufs = ((csrcb, cdstb, gidx, rows, sem),
                        (csrcb2, cdstb2, gidx2, rows2, sem2))

                def fire(wf, bf, buf):
                    csb, cdb, gib, rwb, smb = buf
                    boff = pl.multiple_of(wf * SEG_W + k * SEGCAP + bf * BC,
                                          8)
                    pltpu.sync_copy(bsrc_hbm.at[pl.ds(boff, BC)], csb)
                    pltpu.sync_copy(bdst_hbm.at[pl.ds(boff, BC)], cdb)

                    def gix(v, _):
                        gib[pl.ds(v * 16, 16)] = (
                            csb[pl.ds(v * 16, 16)] + s * NPAD)
                        return 0

                    lax.fori_loop(0, NVB, gix, 0)
                    pltpu.async_copy(y_hbm.at[gib], rwb, smb)

                def consume(buf):
                    csb, cdb, gib, rwb, smb = buf
                    pltpu.make_async_copy(y_hbm.at[gib], rwb, smb).wait()

                    def accv(v, _):
                        ldv = cdb[pl.ds(v * 16, 16)]
                        for l in range(16):
                            plsc.addupdate(acc.at[ldv[l]], rwb[v * 16 + l])
                        return 0

                    lax.fori_loop(0, NVB, accv, 0)

                def adv(wf, bf):
                    b2 = bf + 1
                    roll = b2 >= get_nbq(wf)
                    return (jnp.where(roll, wf + 1, wf),
                            jnp.where(roll, 0, b2))

                fire(0, 0, bufs[0])
                st0 = adv(0, 0)

                def pairbody(i, st):
                    wf, bf = st

                    @pl.when(2 * i + 1 < nbtot)
                    def _():
                        fire(wf, bf, bufs[1])

                    wf2, bf2 = adv(wf, bf)
                    cond1 = 2 * i + 1 < nbtot
                    wf = jnp.where(cond1, wf2, wf)
                    bf = jnp.where(cond1, bf2, bf)
                    consume(bufs[0])

                    @pl.when(2 * i + 2 < nbtot)
                    def _():
                        fire(wf, bf, bufs[0])

                    wf2, bf2 = adv(wf, bf)
                    cond2 = 2 * i + 2 < nbtot
                    wf = jnp.where(cond2, wf2, wf)
                    bf = jnp.where(cond2, bf2, bf)

                    @pl.when(cond1)
                    def _():
                        consume(bufs[1])

                    return (wf, bf)

                lax.fori_loop(0, (nbtot + 1) // 2, pairbody, st0)
            else:
                def prod_body(w, _, j=j):
                    nbq = get_nbq(w)
                    seg = w * SEG_W + k * SEGCAP

                    def bat(b, _):
                        boff = pl.multiple_of(seg + b * BC, 8)

                        @pl.when((b & 15) == s)
                        def _():
                            pltpu.sync_copy(bdst_hbm.at[pl.ds(boff, BC)],
                                            cdstb)

                            def accv(v, _):
                                ldv = cdstb[pl.ds(v * 16, 16)]
                                for l in range(16):
                                    plsc.addupdate(acc.at[ldv[l]], one16)
                                return 0

                            lax.fori_loop(0, NVB, accv, 0)
                        return 0

                    lax.fori_loop(0, nbq, bat, 0)
                    return 0

                lax.fori_loop(0, NW, prod_body, 0)
            pltpu.sync_copy(
                acc, out_hbm.at[pl.ds(pl.multiple_of(s * NPAD + k * R, 8), R)])

    scratch = [
        pltpu.VMEM((BC,), jnp.int32),
        pltpu.VMEM((BC,), jnp.int32),
        pltpu.VMEM((BC,), jnp.int32),
        pltpu.VMEM((BC, 16), jnp.float32),
        pltpu.VMEM((BC,), jnp.int32),
        pltpu.VMEM((BC,), jnp.int32),
        pltpu.VMEM((BC,), jnp.int32),
        pltpu.VMEM((BC, 16), jnp.float32),
        pltpu.VMEM((R, 16), jnp.float32),
        pltpu.VMEM((512,), jnp.int32),
        pltpu.SemaphoreType.DMA,
        pltpu.SemaphoreType.DMA,
    ]
    return pl.kernel(
        body,
        out_type=jax.ShapeDtypeStruct((16 * NPAD, 16), jnp.float32),
        mesh=_mesh,
        scratch_types=scratch,
        compiler_params=pltpu.CompilerParams(use_tc_tiling_on_sc=False),
    )


_prop = _make_prop(True)
_deg_prop = _make_prop(False)


def _dinv_of(deg_ref):
    return lax.rsqrt(deg_ref[:, 0:1] + 1.0)


def _row_mask(i):
    gr = i * BLK + lax.broadcasted_iota(jnp.int32, (BLK, 1), 0)
    return gr < N


def _degsum_body(*refs):
    degs = refs[:16]
    o_ref = refs[16]
    acc = degs[0][...]
    for u in range(1, 16):
        acc = acc + degs[u][...]
    o_ref[...] = acc


_degsum = pl.pallas_call(
    _degsum_body,
    grid=(NBLK,),
    in_specs=[pl.BlockSpec((BLK, 16), lambda i, u=u: (u * NBLK + i, 0))
              for u in range(16)],
    out_specs=pl.BlockSpec((BLK, 16), lambda i: (i, 0)),
    out_shape=jax.ShapeDtypeStruct((NPAD, 16), jnp.float32),
)


_DIMS_NT = (((1,), (1,)), ((), ()))


BLK2 = 2048
NBLK2 = NPAD // BLK2


def _row_mask2(i):
    gr = i * BLK2 + lax.broadcasted_iota(jnp.int32, (BLK2, 1), 0)
    return gr < N


def _prep1_body(x_ref, wt_ref, deg_ref, o_ref):
    i = pl.program_id(1)
    xw = lax.dot_general(x_ref[...], wt_ref[...], _DIMS_NT,
                         preferred_element_type=jnp.float32)
    y = xw * _dinv_of(deg_ref)
    o_ref[...] = jnp.where(_row_mask2(i), y, 0.0)


_prep1 = pl.pallas_call(
    _prep1_body,
    grid=(16, NBLK2),
    in_specs=[pl.BlockSpec((BLK2, 16), lambda t, i: (i, 0)),
              pl.BlockSpec((16, 16), lambda t, i: (t, 0)),
              pl.BlockSpec((BLK2, 16), lambda t, i: (i, 0))],
    out_specs=pl.BlockSpec((BLK2, 16), lambda t, i: (t * NBLK2 + i, 0)),
    out_shape=jax.ShapeDtypeStruct((16 * NPAD, 16), jnp.float32),
)


def _h_body(*refs):
    accs = refs[:16]
    ys = refs[16:32]
    deg_ref, b_ref, o_ref = refs[32:]
    dinv = _dinv_of(deg_ref)
    accf = jnp.concatenate([a[...] for a in accs], axis=1)
    yf = jnp.concatenate([y[...] for y in ys], axis=1)
    o_ref[...] = jax.nn.relu(dinv * (accf + yf) + b_ref[0:1, :])


_hk = pl.pallas_call(
    _h_body,
    grid=(NBLK,),
    in_specs=([pl.BlockSpec((BLK, 16), lambda i, u=u: (u * NBLK + i, 0))
               for u in range(16)]
              + [pl.BlockSpec((BLK, 16), lambda i, u=u: (u * NBLK + i, 0))
                 for u in range(16)]
              + [pl.BlockSpec((BLK, 16), lambda i: (i, 0)),
                 pl.BlockSpec((8, F1), lambda i: (0, 0))]),
    out_specs=pl.BlockSpec((BLK, F1), lambda i: (i, 0)),
    out_shape=jax.ShapeDtypeStruct((NPAD, F1), jnp.float32),
)


def _y_body(h_ref, deg_ref, wt_ref, o_ref):
    i = pl.program_id(1)
    o = lax.dot_general(h_ref[...], wt_ref[...], _DIMS_NT,
                        preferred_element_type=jnp.float32) * _dinv_of(deg_ref)
    o_ref[...] = jnp.where(_row_mask2(i), o, 0.0)


_yk = pl.pallas_call(
    _y_body,
    grid=(16, NBLK2),
    in_specs=[pl.BlockSpec((BLK2, F1), lambda t, i: (i, 0)),
              pl.BlockSpec((BLK2, 16), lambda t, i: (i, 0)),
              pl.BlockSpec((16, F1), lambda t, i: (t, 0))],
    out_specs=pl.BlockSpec((BLK2, 16), lambda t, i: (t * NBLK2 + i, 0)),
    out_shape=jax.ShapeDtypeStruct((16 * NPAD, 16), jnp.float32),
)


def _mid(*args):
    accs_ys = args[:32]
    degc, wt, bp = args[32:]
    h = _hk(*accs_ys, degc, bp)
    return _yk(h, degc, wt)


def _final_body(*refs):
    accs = refs[:16]
    ys = refs[16:32]
    deg_ref, b_ref, batch_ref, sum_ref, cnt_ref = refs[32:]
    i = pl.program_id(0)

    @pl.when(i == 0)
    def _():
        sum_ref[...] = jnp.zeros_like(sum_ref)
        cnt_ref[...] = jnp.zeros_like(cnt_ref)

    dinv = _dinv_of(deg_ref)
    accf = jnp.concatenate([a[...] for a in accs], axis=1)
    yf = jnp.concatenate([y[...] for y in ys], axis=1)
    h = dinv * (accf + yf) + b_ref[0:1, :]
    bb = batch_ref[...].reshape(BLK, 1)
    onehot = (bb == lax.broadcasted_iota(jnp.int32, (BLK, G), 1)).astype(
        jnp.float32)
    dims = (((0,), (0,)), ((), ()))
    sum_ref[...] += lax.dot_general(onehot, h, dims,
                                    preferred_element_type=jnp.float32)
    cnt_ref[...] += lax.dot_general(onehot, jnp.ones((BLK, G), jnp.float32),
                                    dims, preferred_element_type=jnp.float32)

    @pl.when(i == NBLK - 1)
    def _():
        sum_ref[...] = sum_ref[...] / jnp.maximum(cnt_ref[:, 0:1], 1.0)


_final = pl.pallas_call(
    _final_body,
    grid=(NBLK,),
    in_specs=([pl.BlockSpec((BLK, 16), lambda i, u=u: (u * NBLK + i, 0))
               for u in range(16)]
              + [pl.BlockSpec((BLK, 16), lambda i, u=u: (u * NBLK + i, 0))
                 for u in range(16)]
              + [pl.BlockSpec((BLK, 16), lambda i: (i, 0)),
                 pl.BlockSpec((8, F1), lambda i: (0, 0)),
                 pl.BlockSpec((1, 1, BLK), lambda i: (i, 0, 0))]),
    out_specs=[pl.BlockSpec((G, F1), lambda i: (0, 0)),
               pl.BlockSpec((G, G), lambda i: (0, 0))],
    out_shape=[jax.ShapeDtypeStruct((G, F1), jnp.float32),
               jax.ShapeDtypeStruct((G, G), jnp.float32)],
)


def _pad_bias(b):
    return jnp.zeros((8, F1), jnp.float32).at[0].set(b)


def kernel(x, edge_index, batch, W1, b1, W2, b2, W3, b3):
    src = edge_index[0]
    dst = edge_index[1]
    xpad = jnp.zeros((NPAD, 16), jnp.float32).at[:N, :x.shape[1]].set(x)
    w1t = jnp.zeros((F1, 16), jnp.float32).at[:, :W1.shape[0]].set(W1.T)
    w2t = W2.T
    w3t = W3.T
    batch_r = jnp.full((NPAD,), G, jnp.int32).at[:N].set(batch).reshape(
        NBLK, 1, BLK)

    bsrc, bdst, cnts = _bucket(src, dst)
    degp = _deg_prop(bsrc, bdst, cnts)
    degc = _degsum(*([degp] * 16))
    y1 = _prep1(xpad, w1t, degc)
    acc1 = _prop(y1, bsrc, bdst, cnts)
    y2 = _mid(*([acc1] * 16), *([y1] * 16), degc, w2t, _pad_bias(b1))
    acc2 = _prop(y2, bsrc, bdst, cnts)
    y3 = _mid(*([acc2] * 16), *([y2] * 16), degc, w3t, _pad_bias(b2))
    acc3 = _prop(y3, bsrc, bdst, cnts)
    pooled, _ = _final(*([acc3] * 16), *([y3] * 16), degc, _pad_bias(b3),
                       batch_r)
    return pooled
